# baseline (speedup 1.0000x reference)
import functools

import jax
import jax.numpy as jnp
from jax import lax
from jax.experimental import pallas as pl
from jax.experimental.pallas import tpu as pltpu
from jax.experimental.pallas import tpu_sc as plsc

_B, _H, _W, _C = 16, 512, 512, 8
_HW = _H * _W
_NPAD = 512
_NW = 32
_LANES = 16


def _make_gather():
    mesh = plsc.VectorSubcoreMesh(core_axis_name="c", subcore_axis_name="s")

    @functools.partial(
        pl.kernel,
        mesh=mesh,
        out_type=jax.ShapeDtypeStruct((_NW, 2, 128, _C), jnp.float32),
        scratch_types=[
            pltpu.VMEM((2, 128), jnp.int32),
            pltpu.VMEM((2, 128, _C), jnp.float32),
            pltpu.SemaphoreType.DMA,
        ],
        compiler_params=pltpu.CompilerParams(use_tc_tiling_on_sc=False),
    )
    def gather_kernel(table_hbm, idx_hbm, out_hbm, idx_v, rows_v, sem):
        wid = lax.axis_index("s") * 2 + lax.axis_index("c")
        batch = wid // 2
        pltpu.sync_copy(idx_hbm.at[wid], idx_v)
        off = batch * _HW
        for j in range(2):
            for i in range(128 // _LANES):
                sl = pl.ds(i * _LANES, _LANES)
                idx_v[j, sl] = idx_v[j, sl] + off
        cp0 = pltpu.async_copy(table_hbm.at[idx_v.at[0]], rows_v.at[0], sem)
        cp1 = pltpu.async_copy(table_hbm.at[idx_v.at[1]], rows_v.at[1], sem)
        cp0.wait()
        cp1.wait()
        pltpu.sync_copy(rows_v, out_hbm.at[wid])

    return gather_kernel


_gather = _make_gather()


def kernel(feat, index):
    B, H, W, C = feat.shape
    N = index.shape[1]
    table = feat.reshape(B * H * W, C)
    idx = jnp.pad(index, ((0, 0), (0, _NPAD - N))).reshape(_NW, 2, 128)
    out = _gather(table, idx)
    return out.reshape(B, _NPAD, C)[:, :N, :]
